# Initial kernel scaffold; baseline (speedup 1.0000x reference)
import jax
import jax.numpy as jnp
from jax.experimental import pallas as pl


def kernel(edge_index, edge_weight, X, W, b):
    raise NotImplementedError("write your pallas kernel here")


# speedup vs baseline: 2.8217x; 2.8217x over previous
import functools

import jax
import jax.numpy as jnp
from jax import lax
from jax.experimental import pallas as pl
from jax.experimental.pallas import tpu as pltpu
from jax.experimental.pallas import tpu_sc as plsc

N_NODES = 10000
N_EDGES = 320000
D = 128
LANES = 16
NC = 2
NS = 16
NW = NC * NS
EDGES_PER_W = N_EDGES // NW
CHUNK = 80
NCHUNKS = EDGES_PER_W // CHUNK
N_PAD = 10240
ROWS_PER_S = N_PAD // NS
ZROWS = 40


def _sc_scatter(src, dst, w, x):
    mesh = plsc.VectorSubcoreMesh(core_axis_name="c", subcore_axis_name="s")

    @functools.partial(
        pl.kernel,
        mesh=mesh,
        out_type=jax.ShapeDtypeStruct((NC, N_PAD, D), jnp.float32),
        scratch_types=[
            pltpu.VMEM((CHUNK,), jnp.int32),
            pltpu.VMEM((CHUNK,), jnp.int32),
            pltpu.VMEM((CHUNK, LANES), jnp.float32),
            pltpu.VMEM((CHUNK, D), jnp.float32),
            pltpu.VMEM((ZROWS, D), jnp.float32),
            pltpu.VMEM_SHARED((N_PAD, D), jnp.float32),
        ],
    )
    def sc_kernel(src_hbm, dst_hbm, w_hbm, x_hbm, out_hbm,
                  src_v, dst_v, w_v, rows_v, zero_v, h_sh):
        cid = lax.axis_index("c")
        sid = lax.axis_index("s")
        wid = cid * NS + sid

        zv = jnp.zeros((LANES,), jnp.float32)
        for r in range(ZROWS):
            for j in range(D // LANES):
                zero_v[r, pl.ds(j * LANES, LANES)] = zv
        for z in range(ROWS_PER_S // ZROWS):
            pltpu.sync_copy(
                zero_v, h_sh.at[pl.ds(sid * ROWS_PER_S + z * ZROWS, ZROWS)])
        plsc.subcore_barrier()

        def chunk_body(k, carry):
            base = wid * EDGES_PER_W + k * CHUNK
            pltpu.sync_copy(src_hbm.at[pl.ds(base, CHUNK)], src_v)
            pltpu.sync_copy(dst_hbm.at[pl.ds(base, CHUNK)], dst_v)
            pltpu.sync_copy(w_hbm.at[pl.ds(base, CHUNK)], w_v)
            pltpu.sync_copy(x_hbm.at[src_v], rows_v)

            def scale_edge(e, c2):
                wv = w_v[e, :]
                for j in range(D // LANES):
                    sl = pl.ds(j * LANES, LANES)
                    rows_v[e, sl] = rows_v[e, sl] * wv
                return c2

            lax.fori_loop(0, CHUNK, scale_edge, 0)
            pltpu.sync_copy(rows_v, h_sh.at[dst_v], add=True)
            return carry

        lax.fori_loop(0, NCHUNKS, chunk_body, 0)
        plsc.subcore_barrier()

        pltpu.sync_copy(
            h_sh.at[pl.ds(sid * ROWS_PER_S, ROWS_PER_S)],
            out_hbm.at[cid, pl.ds(sid * ROWS_PER_S, ROWS_PER_S)])

    return sc_kernel(src, dst, w, x)


BN = 400


def _tc_body(hp_ref, wt_ref, b_ref, o_ref):
    h = hp_ref[0] + hp_ref[1]
    y = jnp.dot(h, wt_ref[...], preferred_element_type=jnp.float32)
    o_ref[...] = jnp.maximum(y + b_ref[...], 0.0)


def _tc_linear(hp, wt, b):
    return pl.pallas_call(
        _tc_body,
        grid=(N_NODES // BN,),
        in_specs=[
            pl.BlockSpec((NC, BN, D), lambda i: (0, i, 0)),
            pl.BlockSpec((D, D), lambda i: (0, 0)),
            pl.BlockSpec((1, D), lambda i: (0, 0)),
        ],
        out_specs=pl.BlockSpec((BN, D), lambda i: (i, 0)),
        out_shape=jax.ShapeDtypeStruct((N_NODES, D), jnp.float32),
    )(hp, wt, b)


@jax.jit
def kernel(edge_index, edge_weight, X, W, b):
    dst = edge_index[0]
    src = edge_index[1]
    w16 = jnp.broadcast_to(edge_weight[:, None], (N_EDGES, LANES))
    hp = _sc_scatter(src, dst, w16, X)
    return _tc_linear(hp, W.T, b.reshape(1, D))


# speedup vs baseline: 8.9850x; 3.1843x over previous
import functools

import jax
import jax.numpy as jnp
from jax import lax
from jax.experimental import pallas as pl
from jax.experimental.pallas import tpu as pltpu
from jax.experimental.pallas import tpu_sc as plsc

N_NODES = 10000
N_EDGES = 320000
D = 128
LANES = 16
DL = D // LANES
NC = 2
NS = 16
NW = NC * NS
EDGES_PER_W = N_EDGES // NW
CHUNK = 80
NCHUNKS = EDGES_PER_W // CHUNK
NPAIR = NCHUNKS // 2
N_PAD = 10240
ROWS_PER_S = N_PAD // NS
ZROWS = 16
IDX_BITS = 14

_DNUMS = lax.GatherDimensionNumbers(
    offset_dims=(), collapsed_slice_dims=(0,), start_index_map=(0,))


def _bcast_lane(vec, i):
    idx = jnp.full((LANES, 1), i, jnp.int32)
    return lax.gather(vec, idx, _DNUMS, (1,),
                      mode=lax.GatherScatterMode.PROMISE_IN_BOUNDS)


def _sc_scatter(pk_r, w_r, x):
    mesh = plsc.VectorSubcoreMesh(core_axis_name="c", subcore_axis_name="s")

    @functools.partial(
        pl.kernel,
        mesh=mesh,
        out_type=jax.ShapeDtypeStruct((NC, N_PAD, D), jnp.float32),
        scratch_types=[
            pltpu.VMEM((2, CHUNK), jnp.int32),
            pltpu.VMEM((2, CHUNK), jnp.float32),
            pltpu.VMEM((2, CHUNK), jnp.int32),
            pltpu.VMEM((2, CHUNK), jnp.int32),
            pltpu.VMEM((2, CHUNK, D), jnp.float32),
            pltpu.VMEM((ZROWS, D), jnp.float32),
            pltpu.VMEM_SHARED((N_PAD, D), jnp.float32),
            pltpu.SemaphoreType.DMA,
            pltpu.SemaphoreType.DMA,
            pltpu.SemaphoreType.DMA,
            pltpu.SemaphoreType.DMA,
        ],
    )
    def sc_kernel(pk_hbm, w_hbm, x_hbm, out_hbm,
                  pk_v, w_v, src_v, dst_v, rows_v, zero_v, h_sh,
                  psem0, psem1, gsem0, gsem1):
        cid = lax.axis_index("c")
        sid = lax.axis_index("s")
        wid = cid * NS + sid
        psems = (psem0, psem1)
        gsems = (gsem0, gsem1)

        def pkw_issue(k, b):
            pltpu.make_async_copy(
                pk_hbm.at[wid, k], pk_v.at[b], psems[b]).start()
            pltpu.make_async_copy(
                w_hbm.at[wid, k], w_v.at[b], psems[b]).start()

        def pkw_wait(k, b):
            pltpu.make_async_copy(
                pk_hbm.at[wid, k], pk_v.at[b], psems[b]).wait()
            pltpu.make_async_copy(
                w_hbm.at[wid, k], w_v.at[b], psems[b]).wait()

        def decode(b):
            for g in range(CHUNK // LANES):
                sl = pl.ds(g * LANES, LANES)
                p = pk_v[b, sl]
                src_v[b, sl] = lax.bitwise_and(p, (1 << IDX_BITS) - 1)
                dst_v[b, sl] = lax.shift_right_logical(p, IDX_BITS)

        def gather_issue(b):
            pltpu.make_async_copy(
                x_hbm.at[src_v.at[b]], rows_v.at[b], gsems[b]).start()

        def gather_wait(b):
            pltpu.make_async_copy(
                x_hbm.at[src_v.at[b]], rows_v.at[b], gsems[b]).wait()

        def scale(b):
            def group(g, c):
                wg = w_v[b, pl.ds(g * LANES, LANES)]
                for i in range(LANES):
                    e = g * LANES + i
                    wv = _bcast_lane(wg, i)
                    for j in range(DL):
                        sl = pl.ds(j * LANES, LANES)
                        rows_v[b, e, sl] = rows_v[b, e, sl] * wv
                return c
            lax.fori_loop(0, CHUNK // LANES, group, 0)

        def scatter(b):
            pltpu.sync_copy(rows_v.at[b], h_sh.at[dst_v.at[b]], add=True)

        pkw_issue(0, 0)
        pkw_issue(1, 1)

        zv = jnp.zeros((LANES,), jnp.float32)
        for r in range(ZROWS):
            for j in range(DL):
                zero_v[r, pl.ds(j * LANES, LANES)] = zv
        for z in range(ROWS_PER_S // ZROWS):
            pltpu.sync_copy(
                zero_v, h_sh.at[pl.ds(sid * ROWS_PER_S + z * ZROWS, ZROWS)])
        plsc.subcore_barrier()

        pkw_wait(0, 0)
        decode(0)
        gather_issue(0)

        def pair(g, carry):
            k0 = 2 * g
            k1 = k0 + 1
            gather_wait(0)
            pkw_wait(k1, 1)
            decode(1)
            gather_issue(1)
            scale(0)
            pkw_issue(k0 + 2, 0)
            scatter(0)

            gather_wait(1)
            pkw_wait(k0 + 2, 0)
            decode(0)
            gather_issue(0)

            @pl.when(g < NPAIR - 1)
            def _():
                pkw_issue(k1 + 2, 1)

            scale(1)
            scatter(1)
            return carry

        lax.fori_loop(0, NPAIR, pair, 0)

        gather_wait(0)
        scale(0)
        scatter(0)

        plsc.subcore_barrier()

        pltpu.sync_copy(
            h_sh.at[pl.ds(sid * ROWS_PER_S, ROWS_PER_S)],
            out_hbm.at[cid, pl.ds(sid * ROWS_PER_S, ROWS_PER_S)])

    return sc_kernel(pk_r, w_r, x)


BN = 400


def _tc_body(hp_ref, wt_ref, b_ref, o_ref):
    h = hp_ref[0] + hp_ref[1]
    y = jnp.dot(h, wt_ref[...], preferred_element_type=jnp.float32)
    o_ref[...] = jnp.maximum(y + b_ref[...], 0.0)


def _tc_linear(hp, wt, b):
    return pl.pallas_call(
        _tc_body,
        grid=(N_NODES // BN,),
        in_specs=[
            pl.BlockSpec((NC, BN, D), lambda i: (0, i, 0)),
            pl.BlockSpec((D, D), lambda i: (0, 0)),
            pl.BlockSpec((1, D), lambda i: (0, 0)),
        ],
        out_specs=pl.BlockSpec((BN, D), lambda i: (i, 0)),
        out_shape=jax.ShapeDtypeStruct((N_NODES, D), jnp.float32),
    )(hp, wt, b)


@jax.jit
def kernel(edge_index, edge_weight, X, W, b):
    shp = (NW, NCHUNKS, CHUNK)
    pk = (edge_index[0] << IDX_BITS) | edge_index[1]
    pk_r = pk.reshape(shp)
    w_r = edge_weight.reshape(shp)
    hp = _sc_scatter(pk_r, w_r, X)
    return _tc_linear(hp, W.T, b.reshape(1, D))


# speedup vs baseline: 10.7586x; 1.1974x over previous
import functools

import jax
import jax.numpy as jnp
from jax import lax
from jax.experimental import pallas as pl
from jax.experimental.pallas import tpu as pltpu
from jax.experimental.pallas import tpu_sc as plsc

N_NODES = 10000
N_EDGES = 320000
D = 128
LANES = 16
DL = D // LANES
NC = 2
NS = 16
NW = NC * NS
EDGES_PER_W = N_EDGES // NW
CHUNK = 80
NCHUNKS = EDGES_PER_W // CHUNK
NTRIP = NCHUNKS // 3
N_PAD = 10240
ROWS_PER_S = N_PAD // NS
ZROWS = 16
IDX_BITS = 14

_DNUMS = lax.GatherDimensionNumbers(
    offset_dims=(), collapsed_slice_dims=(0,), start_index_map=(0,))


def _bcast_lane(vec, i):
    idx = jnp.full((LANES, 1), i, jnp.int32)
    return lax.gather(vec, idx, _DNUMS, (1,),
                      mode=lax.GatherScatterMode.PROMISE_IN_BOUNDS)


def _sc_scatter(pk_r, w_r, x):
    mesh = plsc.VectorSubcoreMesh(core_axis_name="c", subcore_axis_name="s")

    @functools.partial(
        pl.kernel,
        mesh=mesh,
        out_type=jax.ShapeDtypeStruct((NC, N_PAD, D), jnp.float32),
        scratch_types=[
            pltpu.VMEM((3, CHUNK), jnp.int32),
            pltpu.VMEM((3, CHUNK), jnp.float32),
            pltpu.VMEM((3, CHUNK), jnp.int32),
            pltpu.VMEM((3, CHUNK), jnp.int32),
            pltpu.VMEM((3, CHUNK, D), jnp.float32),
            pltpu.VMEM((ZROWS, D), jnp.float32),
            pltpu.VMEM_SHARED((N_PAD, D), jnp.float32),
            pltpu.SemaphoreType.DMA,
            pltpu.SemaphoreType.DMA,
            pltpu.SemaphoreType.DMA,
            pltpu.SemaphoreType.DMA,
            pltpu.SemaphoreType.DMA,
            pltpu.SemaphoreType.DMA,
            pltpu.SemaphoreType.DMA,
            pltpu.SemaphoreType.DMA,
            pltpu.SemaphoreType.DMA,
        ],
    )
    def sc_kernel(pk_hbm, w_hbm, x_hbm, out_hbm,
                  pk_v, w_v, src_v, dst_v, rows_v, zero_v, h_sh,
                  psem0, psem1, psem2, gsem0, gsem1, gsem2,
                  ssem0, ssem1, ssem2):
        cid = lax.axis_index("c")
        sid = lax.axis_index("s")
        wid = cid * NS + sid
        psems = (psem0, psem1, psem2)
        gsems = (gsem0, gsem1, gsem2)
        ssems = (ssem0, ssem1, ssem2)

        def pkw_issue(k, b):
            pltpu.make_async_copy(
                pk_hbm.at[wid, k], pk_v.at[b], psems[b]).start()
            pltpu.make_async_copy(
                w_hbm.at[wid, k], w_v.at[b], psems[b]).start()

        def pkw_wait(k, b):
            pltpu.make_async_copy(
                pk_hbm.at[wid, k], pk_v.at[b], psems[b]).wait()
            pltpu.make_async_copy(
                w_hbm.at[wid, k], w_v.at[b], psems[b]).wait()

        def decode(b):
            for g in range(CHUNK // LANES):
                sl = pl.ds(g * LANES, LANES)
                p = pk_v[b, sl]
                src_v[b, sl] = lax.bitwise_and(p, (1 << IDX_BITS) - 1)
                dst_v[b, sl] = lax.shift_right_logical(p, IDX_BITS)

        def gather_issue(b):
            pltpu.make_async_copy(
                x_hbm.at[src_v.at[b]], rows_v.at[b], gsems[b]).start()

        def gather_wait(b):
            pltpu.make_async_copy(
                x_hbm.at[src_v.at[b]], rows_v.at[b], gsems[b]).wait()

        def scale(b):
            def group(g, c):
                wg = w_v[b, pl.ds(g * LANES, LANES)]
                for i in range(LANES):
                    e = g * LANES + i
                    wv = _bcast_lane(wg, i)
                    for j in range(DL):
                        sl = pl.ds(j * LANES, LANES)
                        rows_v[b, e, sl] = rows_v[b, e, sl] * wv
                return c
            lax.fori_loop(0, CHUNK // LANES, group, 0)

        def scatter_issue(b):
            pltpu.async_copy(
                rows_v.at[b], h_sh.at[dst_v.at[b]], ssems[b], add=True)

        def scatter_wait(b):
            pltpu.make_async_copy(
                rows_v.at[b], h_sh.at[dst_v.at[b]], ssems[b]).wait()

        def process(k, b, wait_scatter=True, prep=True):
            gather_wait(b)
            scale(b)
            scatter_issue(b)
            if prep:
                @pl.when(k + 3 < NCHUNKS)
                def _():
                    pkw_issue(k + 3, b)
                b2 = (b + 2) % 3
                if wait_scatter:
                    scatter_wait(b2)
                pkw_wait(k + 2, b2)
                decode(b2)
                gather_issue(b2)

        pkw_issue(0, 0)
        pkw_issue(1, 1)
        pkw_issue(2, 2)

        zv = jnp.zeros((LANES,), jnp.float32)
        for r in range(ZROWS):
            for j in range(DL):
                zero_v[r, pl.ds(j * LANES, LANES)] = zv
        for z in range(ROWS_PER_S // ZROWS):
            pltpu.sync_copy(
                zero_v, h_sh.at[pl.ds(sid * ROWS_PER_S + z * ZROWS, ZROWS)])
        plsc.subcore_barrier()

        pkw_wait(0, 0)
        decode(0)
        gather_issue(0)
        pkw_wait(1, 1)
        decode(1)
        gather_issue(1)

        process(0, 0, wait_scatter=False)
        process(1, 1)
        process(2, 2)

        def trip(t, carry):
            k0 = 3 * t
            process(k0, 0)
            process(k0 + 1, 1)
            process(k0 + 2, 2)
            return carry

        lax.fori_loop(1, NTRIP, trip, 0)

        process(NCHUNKS - 2, 0, prep=False)
        process(NCHUNKS - 1, 1, prep=False)
        scatter_wait(2)
        scatter_wait(0)
        scatter_wait(1)

        plsc.subcore_barrier()

        pltpu.sync_copy(
            h_sh.at[pl.ds(sid * ROWS_PER_S, ROWS_PER_S)],
            out_hbm.at[cid, pl.ds(sid * ROWS_PER_S, ROWS_PER_S)])

    return sc_kernel(pk_r, w_r, x)


BN = 400


def _tc_body(hp_ref, wt_ref, b_ref, o_ref):
    h = hp_ref[0] + hp_ref[1]
    y = jnp.dot(h, wt_ref[...], preferred_element_type=jnp.float32)
    o_ref[...] = jnp.maximum(y + b_ref[...], 0.0)


def _tc_linear(hp, wt, b):
    return pl.pallas_call(
        _tc_body,
        grid=(N_NODES // BN,),
        in_specs=[
            pl.BlockSpec((NC, BN, D), lambda i: (0, i, 0)),
            pl.BlockSpec((D, D), lambda i: (0, 0)),
            pl.BlockSpec((1, D), lambda i: (0, 0)),
        ],
        out_specs=pl.BlockSpec((BN, D), lambda i: (i, 0)),
        out_shape=jax.ShapeDtypeStruct((N_NODES, D), jnp.float32),
    )(hp, wt, b)


@jax.jit
def kernel(edge_index, edge_weight, X, W, b):
    shp = (NW, NCHUNKS, CHUNK)
    pk = (edge_index[0] << IDX_BITS) | edge_index[1]
    pk_r = pk.reshape(shp)
    w_r = edge_weight.reshape(shp)
    hp = _sc_scatter(pk_r, w_r, X)
    return _tc_linear(hp, W.T, b.reshape(1, D))
